# baseline (speedup 1.0000x reference)
import functools

import jax
import jax.numpy as jnp
from jax import lax
from jax.experimental import pallas as pl
from jax.experimental.pallas import tpu as pltpu
from jax.experimental.pallas import tpu_sc as plsc

NN = 10000
EE = 320000
GG = 16

NC, NS = 2, 16
CH = 128
EP = 327680
NCHB = EP // CH
TCH = NCHB // NS
CW = 80
WCH = NN // CW
DW = 16
NP = NN + 8

_f32 = jnp.float32


@functools.lru_cache(maxsize=None)
def _make_agg_staged(dh, base_strip):
  K = {16: 8, 32: 8, 48: 4, 64: 2}[dh]
  IB = 16
  NBB = IB // K
  NBI2 = TCH // IB // 2
  mesh = plsc.VectorSubcoreMesh(core_axis_name="c", subcore_axis_name="s")

  @functools.partial(
      pl.kernel, mesh=mesh,
      compiler_params=pltpu.CompilerParams(use_tc_tiling_on_sc=False),
      out_type=jax.ShapeDtypeStruct((NC, NN, dh), _f32),
      scratch_types=[
          pltpu.VMEM((IB, 2, CH), jnp.int32),
          pltpu.VMEM((IB, 2, CH), jnp.int32),
          pltpu.VMEM((K * CH, dh), _f32),
          pltpu.VMEM((K * CH, dh), _f32),
          pltpu.VMEM((CW, dh), _f32),
          pltpu.VMEM_SHARED((NN, dh), _f32),
          pltpu.VMEM_SHARED((NP, dh), _f32),
          pltpu.SemaphoreType.DMA,
          pltpu.SemaphoreType.DMA,
          pltpu.SemaphoreType.DMA,
          pltpu.SemaphoreType.DMA,
          pltpu.SemaphoreType.DMA,
          pltpu.SemaphoreType.DMA,
          pltpu.SemaphoreType.DMA,
      ],
  )
  def agg(u_hbm, ids_hbm, out_hbm,
          ids0, ids1, rows0, rows1, zb, ut, acc,
          sg0, sg1, sc0, sc1, si0, si1, st):
    cid = lax.axis_index("c")
    sid = lax.axis_index("s")

    @pl.loop(0, CW)
    def _(r):
      @pl.loop(0, dh, step=16)
      def _(c0):
        zb[r, pl.ds(c0, 16)] = jnp.zeros((16,), _f32)

    wlo = (sid * WCH) // NS
    whi = ((sid + 1) * WCH) // NS
    u_base = (base_strip + cid) * NN

    def prep(w, carry):
      pltpu.async_copy(u_hbm.at[pl.ds(u_base + w * CW, CW)],
                       ut.at[pl.ds(w * CW, CW)], st)
      pltpu.async_copy(zb, acc.at[pl.ds(w * CW, CW)], sg0)
      return carry

    lax.fori_loop(wlo, whi, prep, 0)

    def prep_wait(w, carry):
      pltpu.make_async_copy(u_hbm.at[pl.ds(u_base + w * CW, CW)],
                            ut.at[pl.ds(w * CW, CW)], st).wait()
      pltpu.make_async_copy(zb, acc.at[pl.ds(w * CW, CW)], sg0).wait()
      return carry

    lax.fori_loop(wlo, whi, prep_wait, 0)

    plsc.subcore_barrier()

    base = sid * TCH
    u_view = u_hbm.at[pl.ds(u_base, NN)]
    gsets = ((rows0, sg0, sc0), (rows1, sg1, sc1))
    isets = ((ids0, si0), (ids1, si1))

    def ids_fetch(ib, si):
      ids, sem = isets[si]
      return pltpu.async_copy(ids_hbm.at[pl.ds(base + ib * IB, IB)], ids, sem)

    def batch(ids):
      gd = [None, None]
      sd = [None, None]

      def fire_g(j):
        rows, sg, _ = gsets[j % 2]
        tab = u_view if j % 4 == 3 else ut
        gd[j % 2] = [
            pltpu.async_copy(tab.at[ids.at[j * K + k, 0]],
                             rows.at[pl.ds(k * CH, CH)], sg)
            for k in range(K)]

      def fire_sc(j):
        rows, _, sc = gsets[j % 2]
        for d in gd[j % 2]:
          d.wait()
        sd[j % 2] = [
            pltpu.async_copy(rows.at[pl.ds(k * CH, CH)],
                             acc.at[ids.at[j * K + k, 1]], sc, add=True)
            for k in range(K)]

      def wait_sc(j):
        for d in sd[j % 2]:
          d.wait()

      fire_g(0)
      for j in range(NBB):
        if j + 1 < NBB:
          if j >= 1:
            wait_sc(j - 1)
          fire_g(j + 1)
        fire_sc(j)
      if NBB >= 2:
        wait_sc(NBB - 2)
      wait_sc(NBB - 1)

    ids_fetch(0, 0).wait()

    @pl.loop(0, NBI2)
    def _(t):
      pf1 = ids_fetch(2 * t + 1, 1)
      batch(ids0)
      pf1.wait()

      @pl.when(t + 1 < NBI2)
      def _():
        ids_fetch(2 * t + 2, 0).wait()
      batch(ids1)

    plsc.subcore_barrier()

    def wb(w, carry):
      pltpu.async_copy(acc.at[pl.ds(w * CW, CW)],
                       out_hbm.at[cid, pl.ds(w * CW, CW)], st)
      return carry

    lax.fori_loop(wlo, whi, wb, 0)

    def wb_wait(w, carry):
      pltpu.make_async_copy(acc.at[pl.ds(w * CW, CW)],
                            out_hbm.at[cid, pl.ds(w * CW, CW)], st).wait()
      return carry

    lax.fori_loop(wlo, whi, wb_wait, 0)

  return agg


def _make_deg():
  mesh = plsc.VectorSubcoreMesh(core_axis_name="c", subcore_axis_name="s")
  KD = 8
  nchunk = NCHB // NC
  tch = nchunk // NS
  NB2 = tch // KD // 2

  @functools.partial(
      pl.kernel, mesh=mesh,
      compiler_params=pltpu.CompilerParams(use_tc_tiling_on_sc=False),
      out_type=jax.ShapeDtypeStruct((NC, NN, DW), _f32),
      scratch_types=[
          pltpu.VMEM((KD, CH), jnp.int32),
          pltpu.VMEM((KD, CH), jnp.int32),
          pltpu.VMEM((CH, DW), _f32),
          pltpu.VMEM((CW, DW), _f32),
          pltpu.VMEM_SHARED((NP, DW), _f32),
          pltpu.SemaphoreType.DMA,
          pltpu.SemaphoreType.DMA,
      ],
  )
  def deg(dst_hbm, out_hbm, idst0, idst1, ones, zb, acc, sc0, sc1):
    cid = lax.axis_index("c")
    sid = lax.axis_index("s")

    one_row = jnp.where(lax.iota(jnp.int32, 16) == 0,
                        jnp.float32(1), jnp.float32(0))

    @pl.loop(0, CH)
    def _(r):
      ones[r, pl.ds(0, 16)] = one_row

    @pl.loop(0, CW)
    def _(r):
      zb[r, pl.ds(0, 16)] = jnp.zeros((16,), _f32)

    wlo = (sid * WCH) // NS
    whi = ((sid + 1) * WCH) // NS

    def zero(w, carry):
      pltpu.sync_copy(zb, acc.at[pl.ds(w * CW, CW)])
      return carry

    lax.fori_loop(wlo, whi, zero, 0)

    plsc.subcore_barrier()

    base = cid * nchunk + sid * tch
    sets = ((idst0, sc0), (idst1, sc1))

    def load_fire(b, si):
      idst, sc = sets[si]
      pltpu.sync_copy(dst_hbm.at[pl.ds(base + b * KD, KD)], idst)
      for j in range(KD):
        pltpu.async_copy(ones, acc.at[idst.at[j]], sc, add=True)

    def wait_sc(si):
      idst, sc = sets[si]
      for j in range(KD):
        pltpu.make_async_copy(ones, acc.at[idst.at[j]], sc).wait()

    load_fire(0, 0)

    @pl.loop(0, NB2)
    def _(t):
      @pl.when(t > 0)
      def _():
        wait_sc(1)
      load_fire(2 * t + 1, 1)

      @pl.when(t + 1 < NB2)
      def _():
        wait_sc(0)
        load_fire(2 * t + 2, 0)

    wait_sc(0)
    wait_sc(1)

    plsc.subcore_barrier()

    def wb(w, carry):
      pltpu.sync_copy(acc.at[pl.ds(w * CW, CW)], zb)
      pltpu.sync_copy(zb, out_hbm.at[cid, pl.ds(w * CW, CW)])
      return carry

    lax.fori_loop(wlo, whi, wb, 0)

  return deg


_deg_kernel = _make_deg()


RB = 2000
NBLK = NN // RB


def _row_spec(w):
  return pl.BlockSpec((RB, w), lambda i: (i, 0))


def _cat_spec(ns, w):
  return pl.BlockSpec((ns, RB, w), lambda i: (0, i, 0))


def _full_spec(shape):
  nd = len(shape)
  return pl.BlockSpec(shape, lambda i: (0,) * nd)


def _act(kind, t):
  if kind == "relu":
    return jnp.maximum(t, 0.0)
  if kind == "lrelu":
    return jnp.where(t > 0, t, 0.01 * t)
  return t


def _dinv_tc(degp):
  def body(p_ref, o_ref):
    d = p_ref[0, :, 0:1] + p_ref[1, :, 0:1] + 1.0
    o_ref[...] = lax.rsqrt(d)

  return pl.pallas_call(
      body,
      grid=(NBLK,),
      in_specs=[_cat_spec(NC, DW)],
      out_specs=_row_spec(1),
      out_shape=jax.ShapeDtypeStruct((NN, 1), _f32),
  )(degp)


def _tc_z(x, W):
  def body(x_ref, w_ref, o_ref):
    o_ref[...] = jnp.dot(x_ref[...], w_ref[...], preferred_element_type=_f32)

  return pl.pallas_call(
      body,
      grid=(NBLK,),
      in_specs=[_row_spec(x.shape[1]), _full_spec(W.shape)],
      out_specs=_row_spec(W.shape[1]),
      out_shape=jax.ShapeDtypeStruct((NN, W.shape[1]), _f32),
  )(x, W)


def _tc_scale(z, dinv, ns):
  dho = z.shape[1] // ns

  def body(z_ref, dv_ref, o_ref):
    u = z_ref[...] * dv_ref[...]
    for k in range(ns):
      o_ref[k] = u[:, k * dho:(k + 1) * dho]

  return pl.pallas_call(
      body,
      grid=(NBLK,),
      in_specs=[_row_spec(z.shape[1]), _row_spec(1)],
      out_specs=_cat_spec(ns, dho),
      out_shape=jax.ShapeDtypeStruct((ns, NN, dho), _f32),
  )(z, dinv)


def _tc_elt(s, u, dinv, b, act):
  ns, _, dh = s.shape

  def body(s_ref, u_ref, dv_ref, b_ref, o_ref):
    dv = dv_ref[...]
    for k in range(ns):
      t = dv * (s_ref[k] + u_ref[k]) + b_ref[0, k * dh:(k + 1) * dh]
      o_ref[k] = dv * _act(act, t)

  return pl.pallas_call(
      body,
      grid=(NBLK,),
      in_specs=[_cat_spec(ns, dh), _cat_spec(ns, dh), _row_spec(1),
                _full_spec((1, ns * dh))],
      out_specs=_cat_spec(ns, dh),
      out_shape=jax.ShapeDtypeStruct((ns, NN, dh), _f32),
  )(s, u, dinv, b.reshape(1, -1))


def _tc_mm(s, u, dinv, W, b, act, nso):
  ns, _, dh = s.shape
  dho = W.shape[1] // nso

  def body(s_ref, u_ref, dv_ref, w_ref, b_ref, o_ref):
    dv = dv_ref[...]
    g = jnp.concatenate([s_ref[k] + u_ref[k] for k in range(ns)], axis=1)
    g = g * dv
    t = jnp.dot(g, w_ref[...], preferred_element_type=_f32) + b_ref[0]
    v = dv * _act(act, t)
    for k in range(nso):
      o_ref[k] = v[:, k * dho:(k + 1) * dho]

  return pl.pallas_call(
      body,
      grid=(NBLK,),
      in_specs=[_cat_spec(ns, dh), _cat_spec(ns, dh), _row_spec(1),
                _full_spec(W.shape), _full_spec((1, W.shape[1]))],
      out_specs=_cat_spec(nso, dho),
      out_shape=jax.ShapeDtypeStruct((nso, NN, dho), _f32),
  )(s, u, dinv, W, b.reshape(1, -1))


def _tc_mm2(s, u, dinv, Wa, ba, act, Wb, nso):
  ns, _, dh = s.shape
  dho = Wb.shape[1] // nso

  def body(s_ref, u_ref, dv_ref, wa_ref, ba_ref, wb_ref, o_ref):
    dv = dv_ref[...]
    g = jnp.concatenate([s_ref[k] + u_ref[k] for k in range(ns)], axis=1)
    g = g * dv
    t = jnp.dot(g, wa_ref[...], preferred_element_type=_f32) + ba_ref[0]
    h = _act(act, t)
    z = jnp.dot(h, wb_ref[...], preferred_element_type=_f32)
    v = dv * z
    for k in range(nso):
      o_ref[k] = v[:, k * dho:(k + 1) * dho]

  return pl.pallas_call(
      body,
      grid=(NBLK,),
      in_specs=[_cat_spec(ns, dh), _cat_spec(ns, dh), _row_spec(1),
                _full_spec(Wa.shape), _full_spec((1, Wa.shape[1])),
                _full_spec(Wb.shape)],
      out_specs=_cat_spec(nso, dho),
      out_shape=jax.ShapeDtypeStruct((nso, NN, dho), _f32),
  )(s, u, dinv, Wa, ba.reshape(1, -1), Wb)


def _tc_elt_mm(s, u, dinv, b, act, W, nso):
  ns, _, dh = s.shape
  dho = W.shape[1] // nso

  def body(s_ref, u_ref, dv_ref, b_ref, w_ref, o_ref):
    dv = dv_ref[...]
    a = jnp.concatenate([s_ref[k] + u_ref[k] for k in range(ns)], axis=1)
    h = _act(act, dv * a + b_ref[0])
    z = jnp.dot(h, w_ref[...], preferred_element_type=_f32)
    v = dv * z
    for k in range(nso):
      o_ref[k] = v[:, k * dho:(k + 1) * dho]

  return pl.pallas_call(
      body,
      grid=(NBLK,),
      in_specs=[_cat_spec(ns, dh), _cat_spec(ns, dh), _row_spec(1),
                _full_spec((1, ns * dh)), _full_spec(W.shape)],
      out_specs=_cat_spec(nso, dho),
      out_shape=jax.ShapeDtypeStruct((nso, NN, dho), _f32),
  )(s, u, dinv, b.reshape(1, -1), W)


def _tc_pool(s, u, dinv, b, batch):
  ns, _, dh = s.shape
  do = ns * dh

  def body(s_ref, u_ref, dv_ref, b_ref, bt_ref, o_ref, acc, cnt):
    i = pl.program_id(0)
    dv = dv_ref[...]
    a = jnp.concatenate([s_ref[k] + u_ref[k] for k in range(ns)], axis=1)
    h = dv * a + b_ref[0]
    gids = lax.broadcasted_iota(jnp.int32, (1, GG), 1)
    sel = (bt_ref[...] == gids).astype(_f32)
    ps = lax.dot_general(sel, h, (((0,), (0,)), ((), ())),
                         preferred_element_type=_f32)
    cs = jnp.sum(sel, axis=0)[:, None] * jnp.ones((1, do), _f32)

    @pl.when(i == 0)
    def _():
      acc[...] = jnp.zeros_like(acc)
      cnt[...] = jnp.zeros_like(cnt)

    acc[...] += ps
    cnt[...] += cs

    @pl.when(i == pl.num_programs(0) - 1)
    def _():
      o_ref[...] = acc[...] / jnp.maximum(cnt[...], 1.0)

  return pl.pallas_call(
      body,
      grid=(NBLK,),
      in_specs=[_cat_spec(ns, dh), _cat_spec(ns, dh), _row_spec(1),
                _full_spec((1, do)), _row_spec(1)],
      out_specs=_full_spec((GG, do)),
      out_shape=jax.ShapeDtypeStruct((GG, do), _f32),
      scratch_shapes=[pltpu.VMEM((GG, do), _f32),
                      pltpu.VMEM((GG, do), _f32)],
  )(s, u, dinv, b.reshape(1, -1), batch.reshape(NN, 1))


def _agg(u_cat, ids):
  ns, _, dh = u_cat.shape
  u2d = u_cat.reshape(ns * NN, dh)
  outs = [_make_agg_staged(dh, k)(u2d, ids) for k in range(0, ns, 2)]
  return outs[0] if len(outs) == 1 else jnp.concatenate(outs, axis=0)


def kernel(x, edge_index, batch,
           W1, b1, W2, b2, W3, b3, W4, b4, W5, b5,
           W6, b6, W7, b7, W8, b8, W9, b9):
  pad = EP - EE
  src2 = jnp.concatenate([edge_index[0],
                          jnp.zeros((pad,), jnp.int32)]).reshape(NCHB, CH)
  dst2 = jnp.concatenate([edge_index[1],
                          jnp.full((pad,), NN, jnp.int32)]).reshape(NCHB, CH)

  ids = jnp.stack([src2, dst2], axis=1)

  z1 = _tc_z(x, W1)
  degp = _deg_kernel(dst2)
  dinv = _dinv_tc(degp)

  u = _tc_scale(z1, dinv, 2)
  s = _agg(u, ids)
  u = _tc_elt(s, u, dinv, b1, "relu")
  s = _agg(u, ids)
  u = _tc_mm(s, u, dinv, W2, b2, "relu", 2)
  s = _agg(u, ids)
  u = _tc_mm(s, u, dinv, W3, b3, "lrelu", 4)
  s = _agg(u, ids)
  u = _tc_mm(s, u, dinv, W4, b4, "relu", 4)
  s = _agg(u, ids)
  u = _tc_mm2(s, u, dinv, W5, b5, "lrelu", W6, 4)
  s = _agg(u, ids)
  u = _tc_elt_mm(s, u, dinv, b6, "lrelu", W7, 2)
  s = _agg(u, ids)
  u = _tc_elt_mm(s, u, dinv, b7, "relu", W8, 2)
  s = _agg(u, ids)
  u = _tc_elt_mm(s, u, dinv, b8, "relu", W9, 2)
  s = _agg(u, ids)
  return _tc_pool(s, u, dinv, b9, batch)


# speedup vs baseline: 1.2215x; 1.2215x over previous
import functools

import jax
import jax.numpy as jnp
from jax import lax
from jax.experimental import pallas as pl
from jax.experimental.pallas import tpu as pltpu
from jax.experimental.pallas import tpu_sc as plsc

NN = 10000
EE = 320000
GG = 16

NC, NS = 2, 16
CH = 128
EP = 327680
NCHB = EP // CH
TCH = NCHB // NS
CW = 80
WCH = NN // CW
DW = 16
NP = NN + 8

_f32 = jnp.float32


@functools.lru_cache(maxsize=None)
def _make_agg_staged(dh, strips):
  K = {16: 8, 32: 8, 48: 4, 64: 2}[dh]
  IB = 16
  NBB = IB // K
  NBI2 = TCH // IB // 2
  mesh = plsc.VectorSubcoreMesh(core_axis_name="c", subcore_axis_name="s")

  @functools.partial(
      pl.kernel, mesh=mesh,
      compiler_params=pltpu.CompilerParams(use_tc_tiling_on_sc=False),
      out_type=jax.ShapeDtypeStruct((len(strips) * NC, NN, dh), _f32),
      scratch_types=[
          pltpu.VMEM((IB, 2, CH), jnp.int32),
          pltpu.VMEM((IB, 2, CH), jnp.int32),
          pltpu.VMEM((K * CH, dh), _f32),
          pltpu.VMEM((K * CH, dh), _f32),
          pltpu.VMEM((CW, dh), _f32),
          pltpu.VMEM_SHARED((NN, dh), _f32),
          pltpu.VMEM_SHARED((NP, dh), _f32),
          pltpu.SemaphoreType.DMA,
          pltpu.SemaphoreType.DMA,
          pltpu.SemaphoreType.DMA,
          pltpu.SemaphoreType.DMA,
          pltpu.SemaphoreType.DMA,
          pltpu.SemaphoreType.DMA,
          pltpu.SemaphoreType.DMA,
      ],
  )
  def agg(u_hbm, ids_hbm, out_hbm,
          ids0, ids1, rows0, rows1, zb, ut, acc,
          sg0, sg1, sc0, sc1, si0, si1, st):
    cid = lax.axis_index("c")
    sid = lax.axis_index("s")

    @pl.loop(0, CW)
    def _(r):
      @pl.loop(0, dh, step=16)
      def _(c0):
        zb[r, pl.ds(c0, 16)] = jnp.zeros((16,), _f32)

    wlo = (sid * WCH) // NS
    whi = ((sid + 1) * WCH) // NS

    def run_pair(p, base_strip):
      u_base = (base_strip + cid) * NN

      def prep(w, carry):
        pltpu.async_copy(u_hbm.at[pl.ds(u_base + w * CW, CW)],
                         ut.at[pl.ds(w * CW, CW)], st)
        pltpu.async_copy(zb, acc.at[pl.ds(w * CW, CW)], sg0)
        return carry

      lax.fori_loop(wlo, whi, prep, 0)

      def prep_wait(w, carry):
        pltpu.make_async_copy(u_hbm.at[pl.ds(u_base + w * CW, CW)],
                              ut.at[pl.ds(w * CW, CW)], st).wait()
        pltpu.make_async_copy(zb, acc.at[pl.ds(w * CW, CW)], sg0).wait()
        return carry

      lax.fori_loop(wlo, whi, prep_wait, 0)

      plsc.subcore_barrier()

      base = sid * TCH
      gsets = ((rows0, sg0, sc0), (rows1, sg1, sc1))
      isets = ((ids0, si0), (ids1, si1))

      def ids_fetch(ib, si):
        ids, sem = isets[si]
        return pltpu.async_copy(ids_hbm.at[pl.ds(base + ib * IB, IB)], ids, sem)

      def batch(ids):
        gd = [None, None]
        sd = [None, None]

        def fire_g(j):
          rows, sg, _ = gsets[j % 2]
          gd[j % 2] = [
              pltpu.async_copy(ut.at[ids.at[j * K + k, 0]],
                               rows.at[pl.ds(k * CH, CH)], sg)
              for k in range(K)]

        def fire_sc(j):
          rows, _, sc = gsets[j % 2]
          for d in gd[j % 2]:
            d.wait()
          sd[j % 2] = [
              pltpu.async_copy(rows.at[pl.ds(k * CH, CH)],
                               acc.at[ids.at[j * K + k, 1]], sc, add=True)
              for k in range(K)]

        def wait_sc(j):
          for d in sd[j % 2]:
            d.wait()

        fire_g(0)
        for j in range(NBB):
          if j + 1 < NBB:
            if j >= 1:
              wait_sc(j - 1)
            fire_g(j + 1)
          fire_sc(j)
        if NBB >= 2:
          wait_sc(NBB - 2)
        wait_sc(NBB - 1)

      ids_fetch(0, 0).wait()

      @pl.loop(0, NBI2)
      def _(t):
        pf1 = ids_fetch(2 * t + 1, 1)
        batch(ids0)
        pf1.wait()

        @pl.when(t + 1 < NBI2)
        def _():
          ids_fetch(2 * t + 2, 0).wait()
        batch(ids1)

      plsc.subcore_barrier()

      def wb(w, carry):
        pltpu.async_copy(acc.at[pl.ds(w * CW, CW)],
                         out_hbm.at[p * NC + cid, pl.ds(w * CW, CW)], st)
        return carry

      lax.fori_loop(wlo, whi, wb, 0)

      def wb_wait(w, carry):
        pltpu.make_async_copy(acc.at[pl.ds(w * CW, CW)],
                              out_hbm.at[p * NC + cid, pl.ds(w * CW, CW)],
                              st).wait()
        return carry

      lax.fori_loop(wlo, whi, wb_wait, 0)

    for p, bs in enumerate(strips):
      run_pair(p, bs)

  return agg


def _make_deg():
  mesh = plsc.VectorSubcoreMesh(core_axis_name="c", subcore_axis_name="s")
  KD = 8
  nchunk = NCHB // NC
  tch = nchunk // NS
  NB2 = tch // KD // 2

  @functools.partial(
      pl.kernel, mesh=mesh,
      compiler_params=pltpu.CompilerParams(use_tc_tiling_on_sc=False),
      out_type=jax.ShapeDtypeStruct((NC, NN, DW), _f32),
      scratch_types=[
          pltpu.VMEM((KD, CH), jnp.int32),
          pltpu.VMEM((KD, CH), jnp.int32),
          pltpu.VMEM((CH, DW), _f32),
          pltpu.VMEM((CW, DW), _f32),
          pltpu.VMEM_SHARED((NP, DW), _f32),
          pltpu.SemaphoreType.DMA,
          pltpu.SemaphoreType.DMA,
      ],
  )
  def deg(dst_hbm, out_hbm, idst0, idst1, ones, zb, acc, sc0, sc1):
    cid = lax.axis_index("c")
    sid = lax.axis_index("s")

    one_row = jnp.where(lax.iota(jnp.int32, 16) == 0,
                        jnp.float32(1), jnp.float32(0))

    @pl.loop(0, CH)
    def _(r):
      ones[r, pl.ds(0, 16)] = one_row

    @pl.loop(0, CW)
    def _(r):
      zb[r, pl.ds(0, 16)] = jnp.zeros((16,), _f32)

    wlo = (sid * WCH) // NS
    whi = ((sid + 1) * WCH) // NS

    def zero(w, carry):
      pltpu.sync_copy(zb, acc.at[pl.ds(w * CW, CW)])
      return carry

    lax.fori_loop(wlo, whi, zero, 0)

    plsc.subcore_barrier()

    base = cid * nchunk + sid * tch
    sets = ((idst0, sc0), (idst1, sc1))

    def load_fire(b, si):
      idst, sc = sets[si]
      pltpu.sync_copy(dst_hbm.at[pl.ds(base + b * KD, KD)], idst)
      for j in range(KD):
        pltpu.async_copy(ones, acc.at[idst.at[j]], sc, add=True)

    def wait_sc(si):
      idst, sc = sets[si]
      for j in range(KD):
        pltpu.make_async_copy(ones, acc.at[idst.at[j]], sc).wait()

    load_fire(0, 0)

    @pl.loop(0, NB2)
    def _(t):
      @pl.when(t > 0)
      def _():
        wait_sc(1)
      load_fire(2 * t + 1, 1)

      @pl.when(t + 1 < NB2)
      def _():
        wait_sc(0)
        load_fire(2 * t + 2, 0)

    wait_sc(0)
    wait_sc(1)

    plsc.subcore_barrier()

    def wb(w, carry):
      pltpu.sync_copy(acc.at[pl.ds(w * CW, CW)], zb)
      pltpu.sync_copy(zb, out_hbm.at[cid, pl.ds(w * CW, CW)])
      return carry

    lax.fori_loop(wlo, whi, wb, 0)

  return deg


_deg_kernel = _make_deg()


RB = 2000
NBLK = NN // RB


def _row_spec(w):
  return pl.BlockSpec((RB, w), lambda i: (i, 0))


def _cat_spec(ns, w):
  return pl.BlockSpec((ns, RB, w), lambda i: (0, i, 0))


def _full_spec(shape):
  nd = len(shape)
  return pl.BlockSpec(shape, lambda i: (0,) * nd)


def _act(kind, t):
  if kind == "relu":
    return jnp.maximum(t, 0.0)
  if kind == "lrelu":
    return jnp.where(t > 0, t, 0.01 * t)
  return t


def _dinv_tc(degp):
  def body(p_ref, o_ref):
    d = p_ref[0, :, 0:1] + p_ref[1, :, 0:1] + 1.0
    o_ref[...] = lax.rsqrt(d)

  return pl.pallas_call(
      body,
      grid=(NBLK,),
      in_specs=[_cat_spec(NC, DW)],
      out_specs=_row_spec(1),
      out_shape=jax.ShapeDtypeStruct((NN, 1), _f32),
  )(degp)


def _tc_z(x, W):
  def body(x_ref, w_ref, o_ref):
    o_ref[...] = jnp.dot(x_ref[...], w_ref[...], preferred_element_type=_f32)

  return pl.pallas_call(
      body,
      grid=(NBLK,),
      in_specs=[_row_spec(x.shape[1]), _full_spec(W.shape)],
      out_specs=_row_spec(W.shape[1]),
      out_shape=jax.ShapeDtypeStruct((NN, W.shape[1]), _f32),
  )(x, W)


def _tc_scale(z, dinv, ns):
  dho = z.shape[1] // ns

  def body(z_ref, dv_ref, o_ref):
    u = z_ref[...] * dv_ref[...]
    for k in range(ns):
      o_ref[k] = u[:, k * dho:(k + 1) * dho]

  return pl.pallas_call(
      body,
      grid=(NBLK,),
      in_specs=[_row_spec(z.shape[1]), _row_spec(1)],
      out_specs=_cat_spec(ns, dho),
      out_shape=jax.ShapeDtypeStruct((ns, NN, dho), _f32),
  )(z, dinv)


def _tc_elt(s, u, dinv, b, act):
  ns, _, dh = s.shape

  def body(s_ref, u_ref, dv_ref, b_ref, o_ref):
    dv = dv_ref[...]
    for k in range(ns):
      t = dv * (s_ref[k] + u_ref[k]) + b_ref[0, k * dh:(k + 1) * dh]
      o_ref[k] = dv * _act(act, t)

  return pl.pallas_call(
      body,
      grid=(NBLK,),
      in_specs=[_cat_spec(ns, dh), _cat_spec(ns, dh), _row_spec(1),
                _full_spec((1, ns * dh))],
      out_specs=_cat_spec(ns, dh),
      out_shape=jax.ShapeDtypeStruct((ns, NN, dh), _f32),
  )(s, u, dinv, b.reshape(1, -1))


def _tc_mm(s, u, dinv, W, b, act, nso):
  ns, _, dh = s.shape
  dho = W.shape[1] // nso

  def body(s_ref, u_ref, dv_ref, w_ref, b_ref, o_ref):
    dv = dv_ref[...]
    g = jnp.concatenate([s_ref[k] + u_ref[k] for k in range(ns)], axis=1)
    g = g * dv
    t = jnp.dot(g, w_ref[...], preferred_element_type=_f32) + b_ref[0]
    v = dv * _act(act, t)
    for k in range(nso):
      o_ref[k] = v[:, k * dho:(k + 1) * dho]

  return pl.pallas_call(
      body,
      grid=(NBLK,),
      in_specs=[_cat_spec(ns, dh), _cat_spec(ns, dh), _row_spec(1),
                _full_spec(W.shape), _full_spec((1, W.shape[1]))],
      out_specs=_cat_spec(nso, dho),
      out_shape=jax.ShapeDtypeStruct((nso, NN, dho), _f32),
  )(s, u, dinv, W, b.reshape(1, -1))


def _tc_mm2(s, u, dinv, Wa, ba, act, Wb, nso):
  ns, _, dh = s.shape
  dho = Wb.shape[1] // nso

  def body(s_ref, u_ref, dv_ref, wa_ref, ba_ref, wb_ref, o_ref):
    dv = dv_ref[...]
    g = jnp.concatenate([s_ref[k] + u_ref[k] for k in range(ns)], axis=1)
    g = g * dv
    t = jnp.dot(g, wa_ref[...], preferred_element_type=_f32) + ba_ref[0]
    h = _act(act, t)
    z = jnp.dot(h, wb_ref[...], preferred_element_type=_f32)
    v = dv * z
    for k in range(nso):
      o_ref[k] = v[:, k * dho:(k + 1) * dho]

  return pl.pallas_call(
      body,
      grid=(NBLK,),
      in_specs=[_cat_spec(ns, dh), _cat_spec(ns, dh), _row_spec(1),
                _full_spec(Wa.shape), _full_spec((1, Wa.shape[1])),
                _full_spec(Wb.shape)],
      out_specs=_cat_spec(nso, dho),
      out_shape=jax.ShapeDtypeStruct((nso, NN, dho), _f32),
  )(s, u, dinv, Wa, ba.reshape(1, -1), Wb)


def _tc_elt_mm(s, u, dinv, b, act, W, nso):
  ns, _, dh = s.shape
  dho = W.shape[1] // nso

  def body(s_ref, u_ref, dv_ref, b_ref, w_ref, o_ref):
    dv = dv_ref[...]
    a = jnp.concatenate([s_ref[k] + u_ref[k] for k in range(ns)], axis=1)
    h = _act(act, dv * a + b_ref[0])
    z = jnp.dot(h, w_ref[...], preferred_element_type=_f32)
    v = dv * z
    for k in range(nso):
      o_ref[k] = v[:, k * dho:(k + 1) * dho]

  return pl.pallas_call(
      body,
      grid=(NBLK,),
      in_specs=[_cat_spec(ns, dh), _cat_spec(ns, dh), _row_spec(1),
                _full_spec((1, ns * dh)), _full_spec(W.shape)],
      out_specs=_cat_spec(nso, dho),
      out_shape=jax.ShapeDtypeStruct((nso, NN, dho), _f32),
  )(s, u, dinv, b.reshape(1, -1), W)


def _tc_pool(s, u, dinv, b, batch):
  ns, _, dh = s.shape
  do = ns * dh

  def body(s_ref, u_ref, dv_ref, b_ref, bt_ref, o_ref, acc, cnt):
    i = pl.program_id(0)
    dv = dv_ref[...]
    a = jnp.concatenate([s_ref[k] + u_ref[k] for k in range(ns)], axis=1)
    h = dv * a + b_ref[0]
    gids = lax.broadcasted_iota(jnp.int32, (1, GG), 1)
    sel = (bt_ref[...] == gids).astype(_f32)
    ps = lax.dot_general(sel, h, (((0,), (0,)), ((), ())),
                         preferred_element_type=_f32)
    cs = jnp.sum(sel, axis=0)[:, None] * jnp.ones((1, do), _f32)

    @pl.when(i == 0)
    def _():
      acc[...] = jnp.zeros_like(acc)
      cnt[...] = jnp.zeros_like(cnt)

    acc[...] += ps
    cnt[...] += cs

    @pl.when(i == pl.num_programs(0) - 1)
    def _():
      o_ref[...] = acc[...] / jnp.maximum(cnt[...], 1.0)

  return pl.pallas_call(
      body,
      grid=(NBLK,),
      in_specs=[_cat_spec(ns, dh), _cat_spec(ns, dh), _row_spec(1),
                _full_spec((1, do)), _row_spec(1)],
      out_specs=_full_spec((GG, do)),
      out_shape=jax.ShapeDtypeStruct((GG, do), _f32),
      scratch_shapes=[pltpu.VMEM((GG, do), _f32),
                      pltpu.VMEM((GG, do), _f32)],
  )(s, u, dinv, b.reshape(1, -1), batch.reshape(NN, 1))


def _agg(u_cat, ids):
  ns, _, dh = u_cat.shape
  u2d = u_cat.reshape(ns * NN, dh)
  return _make_agg_staged(dh, tuple(range(0, ns, 2)))(u2d, ids)


def kernel(x, edge_index, batch,
           W1, b1, W2, b2, W3, b3, W4, b4, W5, b5,
           W6, b6, W7, b7, W8, b8, W9, b9):
  pad = EP - EE
  src2 = jnp.concatenate([edge_index[0],
                          jnp.zeros((pad,), jnp.int32)]).reshape(NCHB, CH)
  dst2 = jnp.concatenate([edge_index[1],
                          jnp.full((pad,), NN, jnp.int32)]).reshape(NCHB, CH)

  ids = jnp.stack([src2, dst2], axis=1)

  z1 = _tc_z(x, W1)
  degp = _deg_kernel(dst2)
  dinv = _dinv_tc(degp)

  u = _tc_scale(z1, dinv, 2)
  s = _agg(u, ids)
  u = _tc_elt(s, u, dinv, b1, "relu")
  s = _agg(u, ids)
  u = _tc_mm(s, u, dinv, W2, b2, "relu", 2)
  s = _agg(u, ids)
  u = _tc_mm(s, u, dinv, W3, b3, "lrelu", 4)
  s = _agg(u, ids)
  u = _tc_mm(s, u, dinv, W4, b4, "relu", 4)
  s = _agg(u, ids)
  u = _tc_mm2(s, u, dinv, W5, b5, "lrelu", W6, 4)
  s = _agg(u, ids)
  u = _tc_elt_mm(s, u, dinv, b6, "lrelu", W7, 2)
  s = _agg(u, ids)
  u = _tc_elt_mm(s, u, dinv, b7, "relu", W8, 2)
  s = _agg(u, ids)
  u = _tc_elt_mm(s, u, dinv, b8, "relu", W9, 2)
  s = _agg(u, ids)
  return _tc_pool(s, u, dinv, b9, batch)


# speedup vs baseline: 1.2240x; 1.0020x over previous
import functools

import jax
import jax.numpy as jnp
from jax import lax
from jax.experimental import pallas as pl
from jax.experimental.pallas import tpu as pltpu
from jax.experimental.pallas import tpu_sc as plsc

NN = 10000
EE = 320000
GG = 16

NC, NS = 2, 16
CH = 128
EP = 327680
NCHB = EP // CH
TCH = NCHB // NS
CW = 80
WCH = NN // CW
DW = 16
NP = NN + 8

_f32 = jnp.float32


@functools.lru_cache(maxsize=None)
def _make_agg_staged(dh, strips):
  K = {16: 8, 32: 8, 48: 4, 64: 2}[dh]
  IB = 16
  NBB = IB // K
  NBI2 = TCH // IB // 2
  mesh = plsc.VectorSubcoreMesh(core_axis_name="c", subcore_axis_name="s")

  @functools.partial(
      pl.kernel, mesh=mesh,
      compiler_params=pltpu.CompilerParams(use_tc_tiling_on_sc=False),
      out_type=jax.ShapeDtypeStruct((len(strips) * NC, NN, dh), _f32),
      scratch_types=[
          pltpu.VMEM((IB, 2, CH), jnp.int32),
          pltpu.VMEM((IB, 2, CH), jnp.int32),
          pltpu.VMEM((K * CH, dh), _f32),
          pltpu.VMEM((K * CH, dh), _f32),
          pltpu.VMEM((CW, dh), _f32),
          pltpu.VMEM_SHARED((NN, dh), _f32),
          pltpu.VMEM_SHARED((NP, dh), _f32),
          pltpu.SemaphoreType.DMA,
          pltpu.SemaphoreType.DMA,
          pltpu.SemaphoreType.DMA,
          pltpu.SemaphoreType.DMA,
          pltpu.SemaphoreType.DMA,
          pltpu.SemaphoreType.DMA,
          pltpu.SemaphoreType.DMA,
      ],
  )
  def agg(u_hbm, ids_hbm, out_hbm,
          ids0, ids1, rows0, rows1, zb, ut, acc,
          sg0, sg1, sc0, sc1, si0, si1, st):
    cid = lax.axis_index("c")
    sid = lax.axis_index("s")

    @pl.loop(0, CW)
    def _(r):
      @pl.loop(0, dh, step=16)
      def _(c0):
        zb[r, pl.ds(c0, 16)] = jnp.zeros((16,), _f32)

    wlo = (sid * WCH) // NS
    whi = ((sid + 1) * WCH) // NS

    def run_pair(p, base_strip):
      u_base = (base_strip + cid) * NN

      def prep(w, carry):
        pltpu.async_copy(u_hbm.at[pl.ds(u_base + w * CW, CW)],
                         ut.at[pl.ds(w * CW, CW)], st)
        pltpu.async_copy(zb, acc.at[pl.ds(w * CW, CW)], sg0)
        return carry

      lax.fori_loop(wlo, whi, prep, 0)

      def prep_wait(w, carry):
        pltpu.make_async_copy(u_hbm.at[pl.ds(u_base + w * CW, CW)],
                              ut.at[pl.ds(w * CW, CW)], st).wait()
        pltpu.make_async_copy(zb, acc.at[pl.ds(w * CW, CW)], sg0).wait()
        return carry

      lax.fori_loop(wlo, whi, prep_wait, 0)

      plsc.subcore_barrier()

      base = sid * TCH
      gsets = ((rows0, sg0, sc0), (rows1, sg1, sc1))
      isets = ((ids0, si0), (ids1, si1))

      def ids_fetch(ib, si):
        ids, sem = isets[si]
        return pltpu.async_copy(ids_hbm.at[pl.ds(base + ib * IB, IB)], ids, sem)

      def batch(ids):
        gd = [None, None]
        sd = [None, None]

        def fire_g(j):
          rows, sg, _ = gsets[j % 2]
          gd[j % 2] = [
              pltpu.async_copy(ut.at[ids.at[j * K + k, 0]],
                               rows.at[pl.ds(k * CH, CH)], sg)
              for k in range(K)]

        def fire_sc(j):
          rows, _, sc = gsets[j % 2]
          for d in gd[j % 2]:
            d.wait()
          sd[j % 2] = [
              pltpu.async_copy(rows.at[pl.ds(k * CH, CH)],
                               acc.at[ids.at[j * K + k, 1]], sc, add=True)
              for k in range(K)]

        def wait_sc(j):
          for d in sd[j % 2]:
            d.wait()

        fire_g(0)
        for j in range(NBB):
          if j + 1 < NBB:
            if j >= 1:
              wait_sc(j - 1)
            fire_g(j + 1)
          fire_sc(j)
        if NBB >= 2:
          wait_sc(NBB - 2)
        wait_sc(NBB - 1)

      ids_fetch(0, 0).wait()

      @pl.loop(0, NBI2)
      def _(t):
        pf1 = ids_fetch(2 * t + 1, 1)
        batch(ids0)
        pf1.wait()

        @pl.when(t + 1 < NBI2)
        def _():
          ids_fetch(2 * t + 2, 0).wait()
        batch(ids1)

      plsc.subcore_barrier()

      def wb(w, carry):
        pltpu.async_copy(acc.at[pl.ds(w * CW, CW)],
                         out_hbm.at[p * NC + cid, pl.ds(w * CW, CW)], st)
        return carry

      lax.fori_loop(wlo, whi, wb, 0)

      def wb_wait(w, carry):
        pltpu.make_async_copy(acc.at[pl.ds(w * CW, CW)],
                              out_hbm.at[p * NC + cid, pl.ds(w * CW, CW)],
                              st).wait()
        return carry

      lax.fori_loop(wlo, whi, wb_wait, 0)

    for p, bs in enumerate(strips):
      run_pair(p, bs)

  return agg


def _make_deg():
  mesh = plsc.VectorSubcoreMesh(core_axis_name="c", subcore_axis_name="s")
  IB = 8
  nchunk = NCHB // NC
  tch = nchunk // NS
  NBI2 = tch // IB // 2

  @functools.partial(
      pl.kernel, mesh=mesh,
      compiler_params=pltpu.CompilerParams(use_tc_tiling_on_sc=False),
      out_type=jax.ShapeDtypeStruct((NC, NN, DW), _f32),
      scratch_types=[
          pltpu.VMEM((IB, 2, CH), jnp.int32),
          pltpu.VMEM((IB, 2, CH), jnp.int32),
          pltpu.VMEM((CH, DW), _f32),
          pltpu.VMEM((CW, DW), _f32),
          pltpu.VMEM_SHARED((NP, DW), _f32),
          pltpu.SemaphoreType.DMA,
          pltpu.SemaphoreType.DMA,
          pltpu.SemaphoreType.DMA,
          pltpu.SemaphoreType.DMA,
      ],
  )
  def deg(ids_hbm, out_hbm, ids0, ids1, ones, zb, acc, sc0, sc1, si0, si1):
    cid = lax.axis_index("c")
    sid = lax.axis_index("s")

    one_row = jnp.where(lax.iota(jnp.int32, 16) == 0,
                        jnp.float32(1), jnp.float32(0))

    @pl.loop(0, CH)
    def _(r):
      ones[r, pl.ds(0, 16)] = one_row

    @pl.loop(0, CW)
    def _(r):
      zb[r, pl.ds(0, 16)] = jnp.zeros((16,), _f32)

    wlo = (sid * WCH) // NS
    whi = ((sid + 1) * WCH) // NS

    def zero(w, carry):
      pltpu.async_copy(zb, acc.at[pl.ds(w * CW, CW)], sc0)
      return carry

    lax.fori_loop(wlo, whi, zero, 0)

    def zero_wait(w, carry):
      pltpu.make_async_copy(zb, acc.at[pl.ds(w * CW, CW)], sc0).wait()
      return carry

    lax.fori_loop(wlo, whi, zero_wait, 0)

    plsc.subcore_barrier()

    base = cid * nchunk + sid * tch
    isets = ((ids0, si0, sc0), (ids1, si1, sc1))

    def ids_fetch(ib, si):
      ids, sem, _ = isets[si]
      return pltpu.async_copy(ids_hbm.at[pl.ds(base + ib * IB, IB)], ids, sem)

    def sbatch(si):
      ids, _, sc = isets[si]
      sd = [pltpu.async_copy(ones, acc.at[ids.at[j, 1]], sc, add=True)
            for j in range(IB)]
      for d in sd:
        d.wait()

    ids_fetch(0, 0).wait()

    @pl.loop(0, NBI2)
    def _(t):
      pf1 = ids_fetch(2 * t + 1, 1)
      sbatch(0)
      pf1.wait()

      @pl.when(t + 1 < NBI2)
      def _():
        ids_fetch(2 * t + 2, 0).wait()
      sbatch(1)

    plsc.subcore_barrier()

    def wb(w, carry):
      pltpu.async_copy(acc.at[pl.ds(w * CW, CW)],
                       out_hbm.at[cid, pl.ds(w * CW, CW)], sc0)
      return carry

    lax.fori_loop(wlo, whi, wb, 0)

    def wb_wait(w, carry):
      pltpu.make_async_copy(acc.at[pl.ds(w * CW, CW)],
                            out_hbm.at[cid, pl.ds(w * CW, CW)], sc0).wait()
      return carry

    lax.fori_loop(wlo, whi, wb_wait, 0)

  return deg


_deg_kernel = _make_deg()


RB = 2000
NBLK = NN // RB


def _row_spec(w):
  return pl.BlockSpec((RB, w), lambda i: (i, 0))


def _cat_spec(ns, w):
  return pl.BlockSpec((ns, RB, w), lambda i: (0, i, 0))


def _full_spec(shape):
  nd = len(shape)
  return pl.BlockSpec(shape, lambda i: (0,) * nd)


def _act(kind, t):
  if kind == "relu":
    return jnp.maximum(t, 0.0)
  if kind == "lrelu":
    return jnp.where(t > 0, t, 0.01 * t)
  return t


def _dinv_scale_tc(degp, z, ns):
  dho = z.shape[1] // ns

  def body(p_ref, z_ref, dv_ref, o_ref):
    d = p_ref[0, :, 0:1] + p_ref[1, :, 0:1] + 1.0
    dv = lax.rsqrt(d)
    dv_ref[...] = dv
    u = z_ref[...] * dv
    for k in range(ns):
      o_ref[k] = u[:, k * dho:(k + 1) * dho]

  return pl.pallas_call(
      body,
      grid=(NBLK,),
      in_specs=[_cat_spec(NC, DW), _row_spec(z.shape[1])],
      out_specs=[_row_spec(1), _cat_spec(ns, dho)],
      out_shape=[jax.ShapeDtypeStruct((NN, 1), _f32),
                 jax.ShapeDtypeStruct((ns, NN, dho), _f32)],
  )(degp, z)


def _tc_z(x, W):
  def body(x_ref, w_ref, o_ref):
    o_ref[...] = jnp.dot(x_ref[...], w_ref[...], preferred_element_type=_f32)

  return pl.pallas_call(
      body,
      grid=(NBLK,),
      in_specs=[_row_spec(x.shape[1]), _full_spec(W.shape)],
      out_specs=_row_spec(W.shape[1]),
      out_shape=jax.ShapeDtypeStruct((NN, W.shape[1]), _f32),
  )(x, W)


def _tc_elt(s, u, dinv, b, act):
  ns, _, dh = s.shape

  def body(s_ref, u_ref, dv_ref, b_ref, o_ref):
    dv = dv_ref[...]
    for k in range(ns):
      t = dv * (s_ref[k] + u_ref[k]) + b_ref[0, k * dh:(k + 1) * dh]
      o_ref[k] = dv * _act(act, t)

  return pl.pallas_call(
      body,
      grid=(NBLK,),
      in_specs=[_cat_spec(ns, dh), _cat_spec(ns, dh), _row_spec(1),
                _full_spec((1, ns * dh))],
      out_specs=_cat_spec(ns, dh),
      out_shape=jax.ShapeDtypeStruct((ns, NN, dh), _f32),
  )(s, u, dinv, b.reshape(1, -1))


def _tc_mm(s, u, dinv, W, b, act, nso):
  ns, _, dh = s.shape
  dho = W.shape[1] // nso

  def body(s_ref, u_ref, dv_ref, w_ref, b_ref, o_ref):
    dv = dv_ref[...]
    g = jnp.concatenate([s_ref[k] + u_ref[k] for k in range(ns)], axis=1)
    g = g * dv
    t = jnp.dot(g, w_ref[...], preferred_element_type=_f32) + b_ref[0]
    v = dv * _act(act, t)
    for k in range(nso):
      o_ref[k] = v[:, k * dho:(k + 1) * dho]

  return pl.pallas_call(
      body,
      grid=(NBLK,),
      in_specs=[_cat_spec(ns, dh), _cat_spec(ns, dh), _row_spec(1),
                _full_spec(W.shape), _full_spec((1, W.shape[1]))],
      out_specs=_cat_spec(nso, dho),
      out_shape=jax.ShapeDtypeStruct((nso, NN, dho), _f32),
  )(s, u, dinv, W, b.reshape(1, -1))


def _tc_mm2(s, u, dinv, Wa, ba, act, Wb, nso):
  ns, _, dh = s.shape
  dho = Wb.shape[1] // nso

  def body(s_ref, u_ref, dv_ref, wa_ref, ba_ref, wb_ref, o_ref):
    dv = dv_ref[...]
    g = jnp.concatenate([s_ref[k] + u_ref[k] for k in range(ns)], axis=1)
    g = g * dv
    t = jnp.dot(g, wa_ref[...], preferred_element_type=_f32) + ba_ref[0]
    h = _act(act, t)
    z = jnp.dot(h, wb_ref[...], preferred_element_type=_f32)
    v = dv * z
    for k in range(nso):
      o_ref[k] = v[:, k * dho:(k + 1) * dho]

  return pl.pallas_call(
      body,
      grid=(NBLK,),
      in_specs=[_cat_spec(ns, dh), _cat_spec(ns, dh), _row_spec(1),
                _full_spec(Wa.shape), _full_spec((1, Wa.shape[1])),
                _full_spec(Wb.shape)],
      out_specs=_cat_spec(nso, dho),
      out_shape=jax.ShapeDtypeStruct((nso, NN, dho), _f32),
  )(s, u, dinv, Wa, ba.reshape(1, -1), Wb)


def _tc_elt_mm(s, u, dinv, b, act, W, nso):
  ns, _, dh = s.shape
  dho = W.shape[1] // nso

  def body(s_ref, u_ref, dv_ref, b_ref, w_ref, o_ref):
    dv = dv_ref[...]
    a = jnp.concatenate([s_ref[k] + u_ref[k] for k in range(ns)], axis=1)
    h = _act(act, dv * a + b_ref[0])
    z = jnp.dot(h, w_ref[...], preferred_element_type=_f32)
    v = dv * z
    for k in range(nso):
      o_ref[k] = v[:, k * dho:(k + 1) * dho]

  return pl.pallas_call(
      body,
      grid=(NBLK,),
      in_specs=[_cat_spec(ns, dh), _cat_spec(ns, dh), _row_spec(1),
                _full_spec((1, ns * dh)), _full_spec(W.shape)],
      out_specs=_cat_spec(nso, dho),
      out_shape=jax.ShapeDtypeStruct((nso, NN, dho), _f32),
  )(s, u, dinv, b.reshape(1, -1), W)


def _tc_pool(s, u, dinv, b, batch):
  ns, _, dh = s.shape
  do = ns * dh

  def body(s_ref, u_ref, dv_ref, b_ref, bt_ref, o_ref, acc, cnt):
    i = pl.program_id(0)
    dv = dv_ref[...]
    a = jnp.concatenate([s_ref[k] + u_ref[k] for k in range(ns)], axis=1)
    h = dv * a + b_ref[0]
    gids = lax.broadcasted_iota(jnp.int32, (1, GG), 1)
    sel = (bt_ref[...] == gids).astype(_f32)
    ps = lax.dot_general(sel, h, (((0,), (0,)), ((), ())),
                         preferred_element_type=_f32)
    cs = jnp.sum(sel, axis=0)[:, None] * jnp.ones((1, do), _f32)

    @pl.when(i == 0)
    def _():
      acc[...] = jnp.zeros_like(acc)
      cnt[...] = jnp.zeros_like(cnt)

    acc[...] += ps
    cnt[...] += cs

    @pl.when(i == pl.num_programs(0) - 1)
    def _():
      o_ref[...] = acc[...] / jnp.maximum(cnt[...], 1.0)

  return pl.pallas_call(
      body,
      grid=(NBLK,),
      in_specs=[_cat_spec(ns, dh), _cat_spec(ns, dh), _row_spec(1),
                _full_spec((1, do)), _row_spec(1)],
      out_specs=_full_spec((GG, do)),
      out_shape=jax.ShapeDtypeStruct((GG, do), _f32),
      scratch_shapes=[pltpu.VMEM((GG, do), _f32),
                      pltpu.VMEM((GG, do), _f32)],
  )(s, u, dinv, b.reshape(1, -1), batch.reshape(NN, 1))


def _agg(u_cat, ids):
  ns, _, dh = u_cat.shape
  u2d = u_cat.reshape(ns * NN, dh)
  return _make_agg_staged(dh, tuple(range(0, ns, 2)))(u2d, ids)


def kernel(x, edge_index, batch,
           W1, b1, W2, b2, W3, b3, W4, b4, W5, b5,
           W6, b6, W7, b7, W8, b8, W9, b9):
  pad = EP - EE
  src2 = jnp.concatenate([edge_index[0],
                          jnp.zeros((pad,), jnp.int32)]).reshape(NCHB, CH)
  dst2 = jnp.concatenate([edge_index[1],
                          jnp.full((pad,), NN, jnp.int32)]).reshape(NCHB, CH)

  ids = jnp.stack([src2, dst2], axis=1)

  z1 = _tc_z(x, W1)
  degp = _deg_kernel(ids)
  dinv, u = _dinv_scale_tc(degp, z1, 2)
  s = _agg(u, ids)
  u = _tc_elt(s, u, dinv, b1, "relu")
  s = _agg(u, ids)
  u = _tc_mm(s, u, dinv, W2, b2, "relu", 2)
  s = _agg(u, ids)
  u = _tc_mm(s, u, dinv, W3, b3, "lrelu", 4)
  s = _agg(u, ids)
  u = _tc_mm(s, u, dinv, W4, b4, "relu", 4)
  s = _agg(u, ids)
  u = _tc_mm2(s, u, dinv, W5, b5, "lrelu", W6, 4)
  s = _agg(u, ids)
  u = _tc_elt_mm(s, u, dinv, b6, "lrelu", W7, 2)
  s = _agg(u, ids)
  u = _tc_elt_mm(s, u, dinv, b7, "relu", W8, 2)
  s = _agg(u, ids)
  u = _tc_elt_mm(s, u, dinv, b8, "relu", W9, 2)
  s = _agg(u, ids)
  return _tc_pool(s, u, dinv, b9, batch)
